# baseline (speedup 1.0000x reference)
import jax
import jax.numpy as jnp
from jax import lax
from jax.experimental import pallas as pl
from jax.experimental.pallas import tpu as pltpu
from jax.experimental.pallas import tpu_sc as plsc

NCHARS = 256
NWORDS = 100000
CHAR_EMB = 8
WORD_EMB = 16
W = 20
S = 50
B = 4096
N = B * S
NC1 = NCHARS + 1
NPAIRS = NC1 * NC1
PAIR_PER_TILE = 4129
NPAIRS_PAD = 16 * PAIR_PER_TILE
WORD_PER_TILE = 6250
NTAB = NPAIRS_PAD + NWORDS + 1
ROWS_PER_POS = 1 + W // 2
OUT_D = WORD_EMB + W * CHAR_EMB

NUM_WORKERS = 32
N_PER = N // NUM_WORKERS
CHUNK = 256
STEPS = N_PER // CHUNK
CROWS = CHUNK * ROWS_PER_POS
STAGE = 2816
LANES = 16


def _build_table(cidx, s, ct_hbm, wt_hbm, tab_hbm, ctv, rows_v, lane):
    tbase = cidx * NTAB
    maskL = lane < CHAR_EMB
    colv = jnp.where(maskL, lane, lane - CHAR_EMB)
    zf = jnp.zeros((LANES,), jnp.float32)
    zi = jnp.zeros((LANES,), jnp.int32)

    pltpu.sync_copy(ct_hbm, ctv)
    plsc.store_scatter(ctv, [zi, lane], zf, mask=maskL)

    def mkrow(c1, c2):
        rowsel = jnp.where(maskL, c1, c2)
        return plsc.load_gather(ctv, [rowsel, colv])

    def pair_loop(r0, nrows, buf, carry):
        def it(r, c):
            c1, c2 = c
            rows_v[buf, r, :] = mkrow(c1, c2)
            wrap = (c2 + 1) == NC1
            return (c1 + wrap.astype(jnp.int32),
                    jnp.where(wrap, 0, c2 + 1))
        return lax.fori_loop(0, nrows, it, carry)

    pr0 = s * PAIR_PER_TILE
    carry = (pr0 // NC1, pr0 % NC1)
    carry = pair_loop(0, STAGE, 0, carry)
    pair_loop(0, PAIR_PER_TILE - STAGE, 1, carry)
    pltpu.sync_copy(rows_v.at[0],
                    tab_hbm.at[pl.ds(tbase + pr0, STAGE), :])
    pltpu.sync_copy(rows_v.at[1, pl.ds(0, PAIR_PER_TILE - STAGE), :],
                    tab_hbm.at[pl.ds(tbase + pr0 + STAGE,
                                     PAIR_PER_TILE - STAGE), :])

    w0 = s * WORD_PER_TILE
    for off, sz in ((0, STAGE), (STAGE, STAGE), (2 * STAGE, WORD_PER_TILE - 2 * STAGE)):
        pltpu.sync_copy(wt_hbm.at[pl.ds(w0 + off, sz), :],
                        rows_v.at[0, pl.ds(0, sz), :])
        if off == 0:
            @pl.when(s == 0)
            def _():
                plsc.store_scatter(rows_v.at[0], [zi, lane], zf)
        pltpu.sync_copy(rows_v.at[0, pl.ds(0, sz), :],
                        tab_hbm.at[pl.ds(tbase + NPAIRS_PAD + w0 + off, sz), :])

    @pl.when(s == 0)
    def _():
        pltpu.sync_copy(wt_hbm.at[pl.ds(NWORDS, 1), :],
                        rows_v.at[0, pl.ds(0, 1), :])
        pltpu.sync_copy(rows_v.at[0, pl.ds(0, 1), :],
                        tab_hbm.at[pl.ds(tbase + NPAIRS_PAD + NWORDS, 1), :])


def _body(cid_hbm, wid_hbm, ct_hbm, wt_hbm, out_hbm, tab_hbm,
          ctv, cid_v, wid_v, idx_v, rows_v,
          sem_l, sem_g, sem_o0, sem_o1):
    ncores = 2
    cidx = lax.axis_index("c")
    s = lax.axis_index("s")
    worker = s * ncores + cidx
    wbase = worker * N_PER
    lane = lax.iota(jnp.int32, LANES)

    _build_table(cidx, s, ct_hbm, wt_hbm, tab_hbm, ctv, rows_v, lane)
    plsc.subcore_barrier()

    wordbase = cidx * NTAB + NPAIRS_PAD
    pairbase = cidx * NTAB

    def loads(k):
        b = k % 2
        return (
            pltpu.make_async_copy(
                cid_hbm.at[pl.ds((wbase + k * CHUNK) * W, CHUNK * W)],
                cid_v.at[b], sem_l),
            pltpu.make_async_copy(
                wid_hbm.at[pl.ds(wbase + k * CHUNK, CHUNK)],
                wid_v.at[b], sem_l),
        )

    def loads_start(k):
        for c in loads(k):
            c.start()

    def loads_wait(k):
        for c in loads(k):
            c.wait()

    def build_idx(k):
        b = k % 2

        def it(i, _):
            pv = i * LANES + lane
            wv = plsc.load_gather(wid_v.at[b], [pv])
            plsc.store_scatter(idx_v.at[b], [pv * ROWS_PER_POS],
                               wv + wordbase)
            cbase = pv * W
            dbase = pv * ROWS_PER_POS + 1
            for q in range(W // 2):
                c1 = plsc.load_gather(cid_v.at[b], [cbase + 2 * q])
                c2 = plsc.load_gather(cid_v.at[b], [cbase + 2 * q + 1])
                plsc.store_scatter(idx_v.at[b], [dbase + q],
                                   c1 * NC1 + c2 + pairbase)
            return ()

        lax.fori_loop(0, CHUNK // LANES, it, ())

    def gather_copy(k):
        return pltpu.make_async_copy(tab_hbm.at[idx_v.at[k % 2]],
                                     rows_v.at[k % 2], sem_g)

    def store_copy(k):
        base = (wbase + k * CHUNK) * ROWS_PER_POS
        sem = sem_o0 if k % 2 == 0 else sem_o1
        return pltpu.make_async_copy(rows_v.at[k % 2],
                                     out_hbm.at[pl.ds(base, CROWS), :], sem)

    loads_start(0)
    if STEPS > 1:
        loads_start(1)
    loads_wait(0)
    build_idx(0)
    gather_copy(0).start()
    if STEPS > 1:
        loads_wait(1)
        build_idx(1)
    if STEPS > 2:
        loads_start(2)

    for k in range(STEPS):
        gather_copy(k).wait()
        store_copy(k).start()
        if k + 1 < STEPS:
            if k >= 1:
                store_copy(k - 1).wait()
            gather_copy(k + 1).start()
        if k + 2 < STEPS:
            loads_wait(k + 2)
            build_idx(k + 2)
        if k + 3 < STEPS:
            loads_start(k + 3)
    if STEPS >= 2:
        store_copy(STEPS - 2).wait()
    store_copy(STEPS - 1).wait()


@jax.jit
def _run(cid_flat, wid_flat, ct, wt):
    mesh = plsc.VectorSubcoreMesh(core_axis_name="c", subcore_axis_name="s")
    out, _ = pl.kernel(
        _body,
        out_type=(
            jax.ShapeDtypeStruct((N * ROWS_PER_POS, WORD_EMB), jnp.float32),
            jax.ShapeDtypeStruct((2 * NTAB, WORD_EMB), jnp.float32),
        ),
        mesh=mesh,
        scratch_types=[
            pltpu.VMEM((NC1, CHAR_EMB), jnp.float32),
            pltpu.VMEM((2, CHUNK * W), jnp.int32),
            pltpu.VMEM((2, CHUNK), jnp.int32),
            pltpu.VMEM((2, CROWS), jnp.int32),
            pltpu.VMEM((2, CROWS, WORD_EMB), jnp.float32),
            pltpu.SemaphoreType.DMA,
            pltpu.SemaphoreType.DMA,
            pltpu.SemaphoreType.DMA,
            pltpu.SemaphoreType.DMA,
        ],
        compiler_params=pltpu.CompilerParams(use_tc_tiling_on_sc=False,
                                             needs_layout_passes=False),
    )(cid_flat, wid_flat, ct, wt)
    return out


def kernel(char_ids, word_ids, char_table, word_table):
    out = _run(char_ids.reshape(N * W), word_ids.reshape(N),
               char_table, word_table)
    return out.reshape(B, S, OUT_D)


# speedup vs baseline: 1.0664x; 1.0664x over previous
import jax
import jax.numpy as jnp
from jax import lax
from jax.experimental import pallas as pl
from jax.experimental.pallas import tpu as pltpu
from jax.experimental.pallas import tpu_sc as plsc

NCHARS = 256
NWORDS = 100000
CHAR_EMB = 8
WORD_EMB = 16
W = 20
S = 50
B = 4096
N = B * S
NC1 = NCHARS + 1
PAIR_PER_TILE = 4129
NPAIRS_PAD = 16 * PAIR_PER_TILE
WORD_PER_TILE = 6250
NTAB = NPAIRS_PAD + NWORDS + 1
ROWS_PER_POS = 1 + W // 2
OUT_D = WORD_EMB + W * CHAR_EMB
DT = OUT_D // 8

NUM_WORKERS = 32
NB = 128
NBT = B // NB
UNITS = S * NBT
U_PER = UNITS // NUM_WORKERS
UROWS = NB * ROWS_PER_POS
STAGE = UROWS
LANES = 16


def _build_table(cidx, s, ct_hbm, wt_hbm, tab_hbm, ctv, rows_v, lane):
    tbase = cidx * NTAB
    maskL = lane < CHAR_EMB
    colv = jnp.where(maskL, lane, lane - CHAR_EMB)
    zf = jnp.zeros((LANES,), jnp.float32)
    zi = jnp.zeros((LANES,), jnp.int32)

    pltpu.sync_copy(ct_hbm, ctv)
    plsc.store_scatter(ctv, [zi, lane], zf, mask=maskL)

    def mkrow(c1, c2):
        rowsel = jnp.where(maskL, c1, c2)
        return plsc.load_gather(ctv, [rowsel, colv])

    def pair_chunk(nrows, carry):
        def it(r, c):
            c1, c2 = c
            rows_v[0, r, :] = mkrow(c1, c2)
            wrap = (c2 + 1) == NC1
            return (c1 + wrap.astype(jnp.int32),
                    jnp.where(wrap, 0, c2 + 1))
        return lax.fori_loop(0, nrows, it, carry)

    pr0 = s * PAIR_PER_TILE
    carry = (pr0 // NC1, pr0 % NC1)
    done = 0
    for sz in (STAGE, STAGE, PAIR_PER_TILE - 2 * STAGE):
        carry = pair_chunk(sz, carry)
        pltpu.sync_copy(rows_v.at[0, pl.ds(0, sz), :],
                        tab_hbm.at[pl.ds(tbase + pr0 + done, sz), :])
        done += sz

    w0 = s * WORD_PER_TILE
    off = 0
    for sz in (STAGE, STAGE, STAGE, STAGE, WORD_PER_TILE - 4 * STAGE):
        pltpu.sync_copy(wt_hbm.at[pl.ds(w0 + off, sz), :],
                        rows_v.at[0, pl.ds(0, sz), :])
        if off == 0:
            @pl.when(s == 0)
            def _():
                plsc.store_scatter(rows_v.at[0], [zi, lane], zf)
        pltpu.sync_copy(rows_v.at[0, pl.ds(0, sz), :],
                        tab_hbm.at[pl.ds(tbase + NPAIRS_PAD + w0 + off, sz), :])
        off += sz

    @pl.when(s == 0)
    def _():
        pltpu.sync_copy(wt_hbm.at[pl.ds(NWORDS, 1), :],
                        rows_v.at[0, pl.ds(0, 1), :])
        pltpu.sync_copy(rows_v.at[0, pl.ds(0, 1), :],
                        tab_hbm.at[pl.ds(tbase + NPAIRS_PAD + NWORDS, 1), :])


def _body(cid_hbm, wid_hbm, ct_hbm, wt_hbm, out_hbm, tab_hbm,
          ctv, cid_v, wid_v, idx_v, rows_v, trans_v,
          sem_l, sem_g, sem_o0, sem_o1):
    ncores = 2
    cidx = lax.axis_index("c")
    s = lax.axis_index("s")
    worker = s * ncores + cidx
    lane = lax.iota(jnp.int32, LANES)
    lane11 = lane * ROWS_PER_POS

    _build_table(cidx, s, ct_hbm, wt_hbm, tab_hbm, ctv, rows_v, lane)
    plsc.subcore_barrier()

    wordbase = cidx * NTAB + NPAIRS_PAD
    pairbase = cidx * NTAB
    ubase = worker * U_PER

    def unit_si_bt(k):
        g = ubase + k
        return g // NBT, g % NBT

    def loads(k, b):
        si, bt = unit_si_bt(k)
        return (
            pltpu.make_async_copy(
                cid_hbm.at[:, si, pl.ds(bt * NB, NB)], cid_v.at[b], sem_l),
            pltpu.make_async_copy(
                wid_hbm.at[si, pl.ds(bt * NB, NB)], wid_v.at[b], sem_l),
        )

    def loads_start(k, b):
        for c in loads(k, b):
            c.start()

    def loads_wait(k, b):
        for c in loads(k, b):
            c.wait()

    def build_idx(k, b):

        def it(i, _):
            pv = i * LANES + lane
            wv = wid_v[b, pl.ds(i * LANES, LANES)]
            plsc.store_scatter(idx_v.at[b], [pv * ROWS_PER_POS],
                               wv + wordbase)
            dbase = pv * ROWS_PER_POS + 1
            for q in range(W // 2):
                c1 = cid_v[b, 2 * q, pl.ds(i * LANES, LANES)]
                c2 = cid_v[b, 2 * q + 1, pl.ds(i * LANES, LANES)]
                plsc.store_scatter(idx_v.at[b], [dbase + q],
                                   c1 * NC1 + c2 + pairbase)
            return ()

        lax.fori_loop(0, NB // LANES, it, ())

    def transpose(k, b):

        def it(d, _):
            dt = d // 8
            dr = d - dt * 8
            rowbase = d // WORD_EMB
            col = d - rowbase * WORD_EMB
            colv = jnp.broadcast_to(col, (LANES,))
            for i in range(NB // LANES):
                rows = lane11 + (i * LANES * ROWS_PER_POS + rowbase)
                v = plsc.load_gather(rows_v.at[b], [rows, colv])
                trans_v[b, dt, dr, pl.ds(i * LANES, LANES)] = v
            return ()

        lax.fori_loop(0, OUT_D, it, ())

    def gather_copy(k, b):
        return pltpu.make_async_copy(tab_hbm.at[idx_v.at[b]],
                                     rows_v.at[b], sem_g)

    def write_copy(k, b):
        si, bt = unit_si_bt(k)
        sem = sem_o0 if b == 0 else sem_o1
        return pltpu.make_async_copy(trans_v.at[b],
                                     out_hbm.at[si, :, bt, :, :], sem)

    def steady(k, b, first=False, gather_next=True, build_next=True,
               loads_next=True, loads_guard=False):
        nb = 1 - b
        gather_copy(k, b).wait()
        if gather_next:
            gather_copy(k + 1, nb).start()
        if not first:
            write_copy(k - 2, b).wait()
        transpose(k, b)
        write_copy(k, b).start()
        if build_next:
            loads_wait(k + 2, b)
            build_idx(k + 2, b)
        if loads_next:
            def start_next():
                loads_start(k + 3, nb)
            if loads_guard:
                pl.when(k + 3 < U_PER)(start_next)
            else:
                start_next()

    loads_start(0, 0)
    loads_start(1, 1)
    loads_wait(0, 0)
    build_idx(0, 0)
    gather_copy(0, 0).start()
    loads_wait(1, 1)
    build_idx(1, 1)
    loads_start(2, 0)
    steady(0, 0, first=True)
    steady(1, 1, first=True)

    def pair_body(g, _):
        k = g * 2
        steady(k, 0, loads_guard=True)
        steady(k + 1, 1, loads_guard=True)
        return ()

    lax.fori_loop(1, U_PER // 2 - 1, pair_body, ())

    steady(U_PER - 2, 0, build_next=False, loads_next=False)
    steady(U_PER - 1, 1, gather_next=False, build_next=False,
           loads_next=False)
    write_copy(U_PER - 2, 0).wait()
    write_copy(U_PER - 1, 1).wait()


@jax.jit
def _run(cid_t, wid_t, ct, wt):
    mesh = plsc.VectorSubcoreMesh(core_axis_name="c", subcore_axis_name="s")
    out, _ = pl.kernel(
        _body,
        out_type=(
            jax.ShapeDtypeStruct((S, DT, NBT, 8, NB), jnp.float32),
            jax.ShapeDtypeStruct((2 * NTAB, WORD_EMB), jnp.float32),
        ),
        mesh=mesh,
        scratch_types=[
            pltpu.VMEM((NC1, CHAR_EMB), jnp.float32),
            pltpu.VMEM((2, W, NB), jnp.int32),
            pltpu.VMEM((2, NB), jnp.int32),
            pltpu.VMEM((2, UROWS), jnp.int32),
            pltpu.VMEM((2, UROWS, WORD_EMB), jnp.float32),
            pltpu.VMEM((2, DT, 8, NB), jnp.float32),
            pltpu.SemaphoreType.DMA,
            pltpu.SemaphoreType.DMA,
            pltpu.SemaphoreType.DMA,
            pltpu.SemaphoreType.DMA,
        ],
        compiler_params=pltpu.CompilerParams(use_tc_tiling_on_sc=False,
                                             needs_layout_passes=False),
    )(cid_t, wid_t, ct, wt)
    return out


def kernel(char_ids, word_ids, char_table, word_table):
    cid_t = char_ids.transpose(2, 1, 0)
    wid_t = word_ids.transpose(1, 0)
    out5 = _run(cid_t, wid_t, char_table, word_table)
    return out5.transpose(2, 4, 0, 1, 3).reshape(B, S, OUT_D)


# speedup vs baseline: 2.7316x; 2.5616x over previous
import jax
import jax.numpy as jnp
from jax import lax
from jax.experimental import pallas as pl
from jax.experimental.pallas import tpu as pltpu
from jax.experimental.pallas import tpu_sc as plsc

NCHARS = 256
NWORDS = 100000
CHAR_EMB = 8
WORD_EMB = 16
W = 20
S = 50
B = 4096
N = B * S
NC1 = NCHARS + 1
PAIR_PER_TILE = 4129
NPAIRS_PAD = 16 * PAIR_PER_TILE
WORD_PER_TILE = 6250
NTAB = NPAIRS_PAD + NWORDS + 1
ROWS_PER_POS = 1 + W // 2
OUT_D = WORD_EMB + W * CHAR_EMB
DT = OUT_D // 8

NUM_WORKERS = 32
NB = 128
NBT = B // NB
UNITS = S * NBT
U_PER = UNITS // NUM_WORKERS
UROWS = NB * ROWS_PER_POS
STAGE = UROWS
LANES = 16


def _build_table(cidx, s, ct_hbm, wt_hbm, tab_hbm, ctv, rows_v, lane):
    tbase = cidx * NTAB
    maskL = lane < CHAR_EMB
    colv = jnp.where(maskL, lane, lane - CHAR_EMB)
    zf = jnp.zeros((LANES,), jnp.float32)
    zi = jnp.zeros((LANES,), jnp.int32)

    pltpu.sync_copy(ct_hbm, ctv)
    plsc.store_scatter(ctv, [zi, lane], zf, mask=maskL)

    def mkrow(c1, c2):
        rowsel = jnp.where(maskL, c1, c2)
        return plsc.load_gather(ctv, [rowsel, colv])

    def pair_chunk(nrows, carry):
        def it(r, c):
            c1, c2 = c
            rows_v[0, r, :] = mkrow(c1, c2)
            wrap = (c2 + 1) == NC1
            return (c1 + wrap.astype(jnp.int32),
                    jnp.where(wrap, 0, c2 + 1))
        return lax.fori_loop(0, nrows, it, carry)

    pr0 = s * PAIR_PER_TILE
    carry = (pr0 // NC1, pr0 % NC1)
    done = 0
    for sz in (STAGE, STAGE, PAIR_PER_TILE - 2 * STAGE):
        carry = pair_chunk(sz, carry)
        pltpu.sync_copy(rows_v.at[0, pl.ds(0, sz), :],
                        tab_hbm.at[pl.ds(tbase + pr0 + done, sz), :])
        done += sz

    w0 = s * WORD_PER_TILE
    off = 0
    for sz in (STAGE, STAGE, STAGE, STAGE, WORD_PER_TILE - 4 * STAGE):
        pltpu.sync_copy(wt_hbm.at[pl.ds(w0 + off, sz), :],
                        rows_v.at[0, pl.ds(0, sz), :])
        if off == 0:
            @pl.when(s == 0)
            def _():
                plsc.store_scatter(rows_v.at[0], [zi, lane], zf)
        pltpu.sync_copy(rows_v.at[0, pl.ds(0, sz), :],
                        tab_hbm.at[pl.ds(tbase + NPAIRS_PAD + w0 + off, sz), :])
        off += sz

    @pl.when(s == 0)
    def _():
        pltpu.sync_copy(wt_hbm.at[pl.ds(NWORDS, 1), :],
                        rows_v.at[0, pl.ds(0, 1), :])
        pltpu.sync_copy(rows_v.at[0, pl.ds(0, 1), :],
                        tab_hbm.at[pl.ds(tbase + NPAIRS_PAD + NWORDS, 1), :])


def _body(cid_hbm, wid_hbm, ct_hbm, wt_hbm, out_hbm, tab_hbm,
          ctv, cid_v, wid_v, idx_v, rows_v, trans_v,
          sem_l, sem_g, sem_o0, sem_o1):
    ncores = 2
    cidx = lax.axis_index("c")
    s = lax.axis_index("s")
    worker = s * ncores + cidx
    lane = lax.iota(jnp.int32, LANES)
    lane11 = lane * ROWS_PER_POS

    _build_table(cidx, s, ct_hbm, wt_hbm, tab_hbm, ctv, rows_v, lane)
    plsc.subcore_barrier()

    wordbase = cidx * NTAB + NPAIRS_PAD
    pairbase = cidx * NTAB
    ubase = worker * U_PER

    def unit_si_bt(k):
        g = ubase + k
        return g // NBT, g % NBT

    def loads(k, b):
        si, bt = unit_si_bt(k)
        return (
            pltpu.make_async_copy(
                cid_hbm.at[:, si, pl.ds(bt * NB, NB)], cid_v.at[b], sem_l),
            pltpu.make_async_copy(
                wid_hbm.at[si, pl.ds(bt * NB, NB)], wid_v.at[b], sem_l),
        )

    def loads_start(k, b):
        for c in loads(k, b):
            c.start()

    def loads_wait(k, b):
        for c in loads(k, b):
            c.wait()

    def build_idx(k, b):

        def it(i):
            pv = i * LANES + lane
            wv = wid_v[b, pl.ds(i * LANES, LANES)]
            plsc.store_scatter(idx_v.at[b], [pv * ROWS_PER_POS],
                               wv + wordbase)
            dbase = pv * ROWS_PER_POS + 1
            for q in range(W // 2):
                c1 = cid_v[b, 2 * q, pl.ds(i * LANES, LANES)]
                c2 = cid_v[b, 2 * q + 1, pl.ds(i * LANES, LANES)]
                plsc.store_scatter(idx_v.at[b], [dbase + q],
                                   c1 * NC1 + c2 + pairbase)

        plsc.parallel_loop(0, NB // LANES, unroll=4)(it)

    def transpose(k, b):

        def it(d):
            dt = d // 8
            dr = d - dt * 8
            rowbase = d // WORD_EMB
            col = d - rowbase * WORD_EMB
            colv = jnp.broadcast_to(col, (LANES,))
            for i in range(NB // LANES):
                rows = lane11 + (i * LANES * ROWS_PER_POS + rowbase)
                v = plsc.load_gather(rows_v.at[b], [rows, colv])
                trans_v[b, dt, dr, pl.ds(i * LANES, LANES)] = v

        plsc.parallel_loop(0, OUT_D, unroll=2)(it)

    def gather_copy(k, b):
        return pltpu.make_async_copy(tab_hbm.at[idx_v.at[b]],
                                     rows_v.at[b], sem_g)

    def write_copy(k, b):
        si, bt = unit_si_bt(k)
        sem = sem_o0 if b == 0 else sem_o1
        return pltpu.make_async_copy(trans_v.at[b],
                                     out_hbm.at[si, :, bt, :, :], sem)

    def steady(k, b, first=False, gather_next=True, build_next=True,
               loads_next=True, loads_guard=False):
        nb = 1 - b
        gather_copy(k, b).wait()
        if gather_next:
            gather_copy(k + 1, nb).start()
        if not first:
            write_copy(k - 2, b).wait()
        transpose(k, b)
        write_copy(k, b).start()
        if build_next:
            loads_wait(k + 2, b)
            build_idx(k + 2, b)
        if loads_next:
            def start_next():
                loads_start(k + 3, nb)
            if loads_guard:
                pl.when(k + 3 < U_PER)(start_next)
            else:
                start_next()

    loads_start(0, 0)
    loads_start(1, 1)
    loads_wait(0, 0)
    build_idx(0, 0)
    gather_copy(0, 0).start()
    loads_wait(1, 1)
    build_idx(1, 1)
    loads_start(2, 0)
    steady(0, 0, first=True)
    steady(1, 1, first=True)

    def pair_body(g, _):
        k = g * 2
        steady(k, 0, loads_guard=True)
        steady(k + 1, 1, loads_guard=True)
        return ()

    lax.fori_loop(1, U_PER // 2 - 1, pair_body, ())

    steady(U_PER - 2, 0, build_next=False, loads_next=False)
    steady(U_PER - 1, 1, gather_next=False, build_next=False,
           loads_next=False)
    write_copy(U_PER - 2, 0).wait()
    write_copy(U_PER - 1, 1).wait()


@jax.jit
def _run(cid_t, wid_t, ct, wt):
    mesh = plsc.VectorSubcoreMesh(core_axis_name="c", subcore_axis_name="s")
    out, _ = pl.kernel(
        _body,
        out_type=(
            jax.ShapeDtypeStruct((S, DT, NBT, 8, NB), jnp.float32),
            jax.ShapeDtypeStruct((2 * NTAB, WORD_EMB), jnp.float32),
        ),
        mesh=mesh,
        scratch_types=[
            pltpu.VMEM((NC1, CHAR_EMB), jnp.float32),
            pltpu.VMEM((2, W, NB), jnp.int32),
            pltpu.VMEM((2, NB), jnp.int32),
            pltpu.VMEM((2, UROWS), jnp.int32),
            pltpu.VMEM((2, UROWS, WORD_EMB), jnp.float32),
            pltpu.VMEM((2, DT, 8, NB), jnp.float32),
            pltpu.SemaphoreType.DMA,
            pltpu.SemaphoreType.DMA,
            pltpu.SemaphoreType.DMA,
            pltpu.SemaphoreType.DMA,
        ],
        compiler_params=pltpu.CompilerParams(use_tc_tiling_on_sc=False,
                                             needs_layout_passes=False),
    )(cid_t, wid_t, ct, wt)
    return out


def kernel(char_ids, word_ids, char_table, word_table):
    cid_t = char_ids.transpose(2, 1, 0)
    wid_t = word_ids.transpose(1, 0)
    out5 = _run(cid_t, wid_t, char_table, word_table)
    return out5.transpose(2, 4, 0, 1, 3).reshape(B, S, OUT_D)


# speedup vs baseline: 2.7424x; 1.0040x over previous
import jax
import jax.numpy as jnp
from jax import lax
from jax.experimental import pallas as pl
from jax.experimental.pallas import tpu as pltpu
from jax.experimental.pallas import tpu_sc as plsc

NCHARS = 256
NWORDS = 100000
CHAR_EMB = 8
WORD_EMB = 16
W = 20
S = 50
B = 4096
N = B * S
NC1 = NCHARS + 1
PAIR_PER_TILE = 4129
NPAIRS_PAD = 16 * PAIR_PER_TILE
WORD_PER_TILE = 6250
NTAB = NPAIRS_PAD + NWORDS + 1
ROWS_PER_POS = 1 + W // 2
OUT_D = WORD_EMB + W * CHAR_EMB
DT = OUT_D // 8

NUM_WORKERS = 32
NB = 128
NBT = B // NB
UNITS = S * NBT
U_PER = UNITS // NUM_WORKERS
UROWS = NB * ROWS_PER_POS
STAGE = UROWS
LANES = 16


def _build_table(cidx, s, ct_hbm, wt_hbm, tab_hbm, ctv, rows_v, lane):
    tbase = cidx * NTAB
    maskL = lane < CHAR_EMB
    colv = jnp.where(maskL, lane, lane - CHAR_EMB)
    zf = jnp.zeros((LANES,), jnp.float32)
    zi = jnp.zeros((LANES,), jnp.int32)

    pltpu.sync_copy(ct_hbm, ctv)
    plsc.store_scatter(ctv, [zi, lane], zf, mask=maskL)

    def mkrow(c1, c2):
        rowsel = jnp.where(maskL, c1, c2)
        return plsc.load_gather(ctv, [rowsel, colv])

    def pair_chunk(nrows, carry):
        def it(r, c):
            c1, c2 = c
            rows_v[0, r, :] = mkrow(c1, c2)
            wrap = (c2 + 1) == NC1
            return (c1 + wrap.astype(jnp.int32),
                    jnp.where(wrap, 0, c2 + 1))
        return lax.fori_loop(0, nrows, it, carry)

    pr0 = s * PAIR_PER_TILE
    carry = (pr0 // NC1, pr0 % NC1)
    done = 0
    for sz in (STAGE, STAGE, PAIR_PER_TILE - 2 * STAGE):
        carry = pair_chunk(sz, carry)
        pltpu.sync_copy(rows_v.at[0, pl.ds(0, sz), :],
                        tab_hbm.at[pl.ds(tbase + pr0 + done, sz), :])
        done += sz

    w0 = s * WORD_PER_TILE
    off = 0
    for sz in (STAGE, STAGE, STAGE, STAGE, WORD_PER_TILE - 4 * STAGE):
        pltpu.sync_copy(wt_hbm.at[pl.ds(w0 + off, sz), :],
                        rows_v.at[0, pl.ds(0, sz), :])
        if off == 0:
            @pl.when(s == 0)
            def _():
                plsc.store_scatter(rows_v.at[0], [zi, lane], zf)
        pltpu.sync_copy(rows_v.at[0, pl.ds(0, sz), :],
                        tab_hbm.at[pl.ds(tbase + NPAIRS_PAD + w0 + off, sz), :])
        off += sz

    @pl.when(s == 0)
    def _():
        pltpu.sync_copy(wt_hbm.at[pl.ds(NWORDS, 1), :],
                        rows_v.at[0, pl.ds(0, 1), :])
        pltpu.sync_copy(rows_v.at[0, pl.ds(0, 1), :],
                        tab_hbm.at[pl.ds(tbase + NPAIRS_PAD + NWORDS, 1), :])


def _body(cid_hbm, wid_hbm, ct_hbm, wt_hbm, out_hbm, tab_hbm,
          ctv, cid_v, wid_v, idx_v, rows_v, trans_v,
          sem_l, sem_g, sem_o0, sem_o1):
    ncores = 2
    cidx = lax.axis_index("c")
    s = lax.axis_index("s")
    worker = s * ncores + cidx
    lane = lax.iota(jnp.int32, LANES)
    lane11 = lane * ROWS_PER_POS

    _build_table(cidx, s, ct_hbm, wt_hbm, tab_hbm, ctv, rows_v, lane)
    plsc.subcore_barrier()

    wordbase = cidx * NTAB + NPAIRS_PAD
    pairbase = cidx * NTAB
    ubase = worker * U_PER

    def unit_si_bt(k):
        g = ubase + k
        return g // NBT, g % NBT

    def loads(k, b):
        si, bt = unit_si_bt(k)
        return (
            pltpu.make_async_copy(
                cid_hbm.at[:, si, pl.ds(bt * NB, NB)], cid_v.at[b], sem_l),
            pltpu.make_async_copy(
                wid_hbm.at[si, pl.ds(bt * NB, NB)], wid_v.at[b], sem_l),
        )

    def loads_start(k, b):
        for c in loads(k, b):
            c.start()

    def loads_wait(k, b):
        for c in loads(k, b):
            c.wait()

    def build_idx(k, b):

        def it(i):
            pv = i * LANES + lane
            wv = wid_v[b, pl.ds(i * LANES, LANES)]
            plsc.store_scatter(idx_v.at[b], [pv * ROWS_PER_POS],
                               wv + wordbase)
            dbase = pv * ROWS_PER_POS + 1
            for q in range(W // 2):
                c1 = cid_v[b, 2 * q, pl.ds(i * LANES, LANES)]
                c2 = cid_v[b, 2 * q + 1, pl.ds(i * LANES, LANES)]
                plsc.store_scatter(idx_v.at[b], [dbase + q],
                                   c1 * NC1 + c2 + pairbase)

        plsc.parallel_loop(0, NB // LANES, unroll=4)(it)

    def transpose(k, b):

        def it(d):
            dt = d // 8
            dr = d - dt * 8
            rowbase = d // WORD_EMB
            col = d - rowbase * WORD_EMB
            colv = jnp.broadcast_to(col, (LANES,))
            for i in range(NB // LANES):
                rows = lane11 + (i * LANES * ROWS_PER_POS + rowbase)
                v = plsc.load_gather(rows_v.at[b], [rows, colv])
                trans_v[b, dt, dr, pl.ds(i * LANES, LANES)] = v

        plsc.parallel_loop(0, OUT_D, unroll=4)(it)

    def gather_copy(k, b):
        return pltpu.make_async_copy(tab_hbm.at[idx_v.at[b]],
                                     rows_v.at[b], sem_g)

    def write_copy(k, b):
        si, bt = unit_si_bt(k)
        sem = sem_o0 if b == 0 else sem_o1
        return pltpu.make_async_copy(trans_v.at[b],
                                     out_hbm.at[si, :, bt, :, :], sem)

    def steady(k, b, first=False, gather_next=True, build_next=True,
               loads_next=True, loads_guard=False):
        nb = 1 - b
        gather_copy(k, b).wait()
        if gather_next:
            gather_copy(k + 1, nb).start()
        if not first:
            write_copy(k - 2, b).wait()
        transpose(k, b)
        write_copy(k, b).start()
        if build_next:
            loads_wait(k + 2, b)
            build_idx(k + 2, b)
        if loads_next:
            def start_next():
                loads_start(k + 3, nb)
            if loads_guard:
                pl.when(k + 3 < U_PER)(start_next)
            else:
                start_next()

    loads_start(0, 0)
    loads_start(1, 1)
    loads_wait(0, 0)
    build_idx(0, 0)
    gather_copy(0, 0).start()
    loads_wait(1, 1)
    build_idx(1, 1)
    loads_start(2, 0)
    steady(0, 0, first=True)
    steady(1, 1, first=True)

    def pair_body(g, _):
        k = g * 2
        steady(k, 0, loads_guard=True)
        steady(k + 1, 1, loads_guard=True)
        return ()

    lax.fori_loop(1, U_PER // 2 - 1, pair_body, ())

    steady(U_PER - 2, 0, build_next=False, loads_next=False)
    steady(U_PER - 1, 1, gather_next=False, build_next=False,
           loads_next=False)
    write_copy(U_PER - 2, 0).wait()
    write_copy(U_PER - 1, 1).wait()


@jax.jit
def _run(cid_t, wid_t, ct, wt):
    mesh = plsc.VectorSubcoreMesh(core_axis_name="c", subcore_axis_name="s")
    out, _ = pl.kernel(
        _body,
        out_type=(
            jax.ShapeDtypeStruct((S, DT, NBT, 8, NB), jnp.float32),
            jax.ShapeDtypeStruct((2 * NTAB, WORD_EMB), jnp.float32),
        ),
        mesh=mesh,
        scratch_types=[
            pltpu.VMEM((NC1, CHAR_EMB), jnp.float32),
            pltpu.VMEM((2, W, NB), jnp.int32),
            pltpu.VMEM((2, NB), jnp.int32),
            pltpu.VMEM((2, UROWS), jnp.int32),
            pltpu.VMEM((2, UROWS, WORD_EMB), jnp.float32),
            pltpu.VMEM((2, DT, 8, NB), jnp.float32),
            pltpu.SemaphoreType.DMA,
            pltpu.SemaphoreType.DMA,
            pltpu.SemaphoreType.DMA,
            pltpu.SemaphoreType.DMA,
        ],
        compiler_params=pltpu.CompilerParams(use_tc_tiling_on_sc=False,
                                             needs_layout_passes=False),
    )(cid_t, wid_t, ct, wt)
    return out


def kernel(char_ids, word_ids, char_table, word_table):
    cid_t = char_ids.transpose(2, 1, 0)
    wid_t = word_ids.transpose(1, 0)
    out5 = _run(cid_t, wid_t, char_table, word_table)
    return out5.transpose(2, 4, 0, 1, 3).reshape(B, S, OUT_D)
